# baseline (speedup 1.0000x reference)
import functools

import jax
import jax.numpy as jnp
from jax import lax
from jax.experimental import pallas as pl
from jax.experimental.pallas import tpu as pltpu
from jax.experimental.pallas import tpu_sc as plsc
from jax.experimental.layout import Layout, with_layout_constraint

BATCH = 16384
DIM = 64
NC = 2
NS = 16
NW = NC * NS
BPW = BATCH // NW
L = 16
NG = BPW // L


def _rsqrt16(x):
    i = plsc.bitcast(x, jnp.int32)
    i = jnp.int32(0x5F3759DF) - lax.shift_right_arithmetic(i, jnp.int32(1))
    y = plsc.bitcast(i, jnp.float32)
    xh = x * jnp.float32(0.5)
    for _ in range(3):
        y = y * (jnp.float32(1.5) - xh * y * y)
    return y


CCH = 256
NCC = BPW // CCH
NGC = CCH // L


HALF = 500000


def _tile_kernel(heads, rels, tails, ent_lo, ent_hi, rel, out,
                 ih, ir, it, hr, rr, tr, sc, sem):
    wid = lax.axis_index("s") * NC + lax.axis_index("c")
    base = wid * BPW

    pltpu.sync_copy(heads.at[pl.ds(base, BPW)], ih)
    pltpu.sync_copy(rels.at[pl.ds(base, BPW)], ir)
    pltpu.sync_copy(tails.at[pl.ds(base, BPW)], it)

    iota = lax.iota(jnp.int32, L)
    zero = jnp.zeros((L,), jnp.float32)

    for ck in range(NCC):
        def fetch(g, ck=ck):
            rb = g * L
            gb = ck * CCH + rb
            vh = ih[pl.ds(gb, L)]
            vr = ir[pl.ds(gb, L)]
            vt = it[pl.ds(gb, L)]
            copies = []
            for k in range(L):
                dst = (pl.ds(rb + k, 1), pl.ds(0, DIM))

                def fire_ent(idx, dbuf, d0=dst[0], d1=dst[1]):
                    @pl.when(idx < HALF)
                    def _():
                        pltpu.async_copy(
                            ent_lo.at[pl.ds(idx, 1), :], dbuf.at[d0, d1], sem)

                    @pl.when(idx >= HALF)
                    def _():
                        pltpu.async_copy(
                            ent_hi.at[pl.ds(idx - HALF, 1), :],
                            dbuf.at[d0, d1], sem)

                    copies.append(pltpu.make_async_copy(
                        ent_lo.at[pl.ds(0, 1), :], dbuf.at[d0, d1], sem))

                fire_ent(vh[k], hr)
                copies.append(pltpu.async_copy(
                    rel.at[pl.ds(vr[k], 1), :], rr.at[dst[0], dst[1]], sem))
                fire_ent(vt[k], tr)
            return copies

        def group(g, _, ck=ck):
            rowi = g * L + iota

            sh0 = zero
            sh1 = zero
            st0 = zero
            st1 = zero
            for j in range(0, DIM, 2):
                c0 = jnp.full((L,), j, jnp.int32)
                c1 = jnp.full((L,), j + 1, jnp.int32)
                hv0 = plsc.load_gather(hr, [rowi, c0])
                hv1 = plsc.load_gather(hr, [rowi, c1])
                tv0 = plsc.load_gather(tr, [rowi, c0])
                tv1 = plsc.load_gather(tr, [rowi, c1])
                sh0 = sh0 + hv0 * hv0
                sh1 = sh1 + hv1 * hv1
                st0 = st0 + tv0 * tv0
                st1 = st1 + tv1 * tv1
            invh = _rsqrt16(sh0 + sh1)
            invt = _rsqrt16(st0 + st1)

            a0 = zero
            a1 = zero
            for j in range(0, DIM, 2):
                c0 = jnp.full((L,), j, jnp.int32)
                c1 = jnp.full((L,), j + 1, jnp.int32)
                hv0 = plsc.load_gather(hr, [rowi, c0])
                hv1 = plsc.load_gather(hr, [rowi, c1])
                tv0 = plsc.load_gather(tr, [rowi, c0])
                tv1 = plsc.load_gather(tr, [rowi, c1])
                rv0 = plsc.load_gather(rr, [rowi, c0])
                rv1 = plsc.load_gather(rr, [rowi, c1])
                a0 = a0 + jnp.abs(hv0 * invh + rv0 - tv0 * invt)
                a1 = a1 + jnp.abs(hv1 * invh + rv1 - tv1 * invt)
            sc[pl.ds(ck * CCH + g * L, L)] = a0 + a1
            return 0

        def step(g, _, ck=ck):
            copies = fetch(g, ck=ck)

            @pl.when(g > 0)
            def _():
                group(g - 1, None, ck=ck)

            for cp in copies:
                cp.wait()
            return 0

        lax.fori_loop(0, NGC, step, 0)
        group(NGC - 1, None, ck=ck)

    pltpu.sync_copy(sc, out.at[pl.ds(base, BPW)])


@jax.jit
def _transe_sc(heads, rels, tails, ent, rel):
    ent_lo = with_layout_constraint(
        ent[:HALF], Layout(major_to_minor=(1, 0), tiling=((8, 128),)))
    ent_hi = with_layout_constraint(
        ent[HALF:], Layout(major_to_minor=(1, 0), tiling=((8, 128),)))
    rel = with_layout_constraint(
        rel, Layout(major_to_minor=(1, 0), tiling=((8, 128),)))
    mesh = plsc.VectorSubcoreMesh(core_axis_name="c", subcore_axis_name="s")
    f = functools.partial(
        pl.kernel,
        mesh=mesh,
        out_type=jax.ShapeDtypeStruct((BATCH,), jnp.float32),
        scratch_types=[
            pltpu.VMEM((BPW,), jnp.int32),
            pltpu.VMEM((BPW,), jnp.int32),
            pltpu.VMEM((BPW,), jnp.int32),
            pltpu.VMEM((CCH, DIM), jnp.float32),
            pltpu.VMEM((CCH, DIM), jnp.float32),
            pltpu.VMEM((CCH, DIM), jnp.float32),
            pltpu.VMEM((BPW,), jnp.float32),
            pltpu.SemaphoreType.DMA,
        ],
        compiler_params=pltpu.CompilerParams(
            needs_layout_passes=False,
        ),
    )(_tile_kernel)
    return f(heads, rels, tails, ent_lo, ent_hi, rel)


def kernel(heads, relations, tails, entity_emb, relation_emb):
    heads = jnp.asarray(heads, jnp.int32)
    relations = jnp.asarray(relations, jnp.int32)
    tails = jnp.asarray(tails, jnp.int32)
    return _transe_sc(heads, relations, tails, entity_emb, relation_emb)


# speedup vs baseline: 1.3019x; 1.3019x over previous
import functools

import jax
import jax.numpy as jnp
from jax import lax
from jax.experimental import pallas as pl
from jax.experimental.pallas import tpu as pltpu
from jax.experimental.pallas import tpu_sc as plsc
from jax.experimental.layout import Layout, with_layout_constraint

BATCH = 16384
DIM = 64
NC = 2
NS = 16
NW = NC * NS
BPW = BATCH // NW
L = 16
NG = BPW // L


def _rsqrt16(x):
    i = plsc.bitcast(x, jnp.int32)
    i = jnp.int32(0x5F3759DF) - lax.shift_right_arithmetic(i, jnp.int32(1))
    y = plsc.bitcast(i, jnp.float32)
    xh = x * jnp.float32(0.5)
    for _ in range(3):
        y = y * (jnp.float32(1.5) - xh * y * y)
    return y


CCH = 256
NCC = BPW // CCH
NGC = CCH // L


def _tile_kernel(heads, rels, tails, ent, rel, out,
                 ih, ir, it, hr, rr, tr, sc, sem):
    wid = lax.axis_index("s") * NC + lax.axis_index("c")
    base = wid * BPW

    pltpu.sync_copy(heads.at[pl.ds(base, BPW)], ih)
    pltpu.sync_copy(rels.at[pl.ds(base, BPW)], ir)
    pltpu.sync_copy(tails.at[pl.ds(base, BPW)], it)

    iota = lax.iota(jnp.int32, L)
    zero = jnp.zeros((L,), jnp.float32)

    for ck in range(NCC):
        def fetch(g, ck=ck):
            rb = g * L
            gb = ck * CCH + rb
            vh = ih[pl.ds(gb, L)]
            vr = ir[pl.ds(gb, L)]
            vt = it[pl.ds(gb, L)]
            copies = []
            for k in range(L):
                dst = (pl.ds(rb + k, 1), pl.ds(0, DIM))
                copies.append(pltpu.async_copy(
                    ent.at[pl.ds(vh[k], 1), :], hr.at[dst[0], dst[1]], sem))
                copies.append(pltpu.async_copy(
                    rel.at[pl.ds(vr[k], 1), :], rr.at[dst[0], dst[1]], sem))
                copies.append(pltpu.async_copy(
                    ent.at[pl.ds(vt[k], 1), :], tr.at[dst[0], dst[1]], sem))
            return copies

        def group(g, _, ck=ck):
            rowi = g * L + iota

            sh0 = zero
            sh1 = zero
            st0 = zero
            st1 = zero
            for j in range(0, DIM, 2):
                c0 = jnp.full((L,), j, jnp.int32)
                c1 = jnp.full((L,), j + 1, jnp.int32)
                hv0 = plsc.load_gather(hr, [rowi, c0])
                hv1 = plsc.load_gather(hr, [rowi, c1])
                tv0 = plsc.load_gather(tr, [rowi, c0])
                tv1 = plsc.load_gather(tr, [rowi, c1])
                sh0 = sh0 + hv0 * hv0
                sh1 = sh1 + hv1 * hv1
                st0 = st0 + tv0 * tv0
                st1 = st1 + tv1 * tv1
            invh = _rsqrt16(sh0 + sh1)
            invt = _rsqrt16(st0 + st1)

            a0 = zero
            a1 = zero
            for j in range(0, DIM, 2):
                c0 = jnp.full((L,), j, jnp.int32)
                c1 = jnp.full((L,), j + 1, jnp.int32)
                hv0 = plsc.load_gather(hr, [rowi, c0])
                hv1 = plsc.load_gather(hr, [rowi, c1])
                tv0 = plsc.load_gather(tr, [rowi, c0])
                tv1 = plsc.load_gather(tr, [rowi, c1])
                rv0 = plsc.load_gather(rr, [rowi, c0])
                rv1 = plsc.load_gather(rr, [rowi, c1])
                a0 = a0 + jnp.abs(hv0 * invh + rv0 - tv0 * invt)
                a1 = a1 + jnp.abs(hv1 * invh + rv1 - tv1 * invt)
            sc[pl.ds(ck * CCH + g * L, L)] = a0 + a1
            return 0

        def step(g, _, ck=ck):
            copies = fetch(g, ck=ck)

            @pl.when(g > 0)
            def _():
                group(g - 1, None, ck=ck)

            for cp in copies:
                cp.wait()
            return 0

        lax.fori_loop(0, NGC, step, 0)
        group(NGC - 1, None, ck=ck)

    pltpu.sync_copy(sc, out.at[pl.ds(base, BPW)])


@jax.jit
def _transe_sc(heads, rels, tails, ent, rel):
    ent = with_layout_constraint(
        ent, Layout(major_to_minor=(1, 0), tiling=((8, 128),)))
    rel = with_layout_constraint(
        rel, Layout(major_to_minor=(1, 0), tiling=((8, 128),)))
    mesh = plsc.VectorSubcoreMesh(core_axis_name="c", subcore_axis_name="s")
    f = functools.partial(
        pl.kernel,
        mesh=mesh,
        out_type=jax.ShapeDtypeStruct((BATCH,), jnp.float32),
        scratch_types=[
            pltpu.VMEM((BPW,), jnp.int32),
            pltpu.VMEM((BPW,), jnp.int32),
            pltpu.VMEM((BPW,), jnp.int32),
            pltpu.VMEM((CCH, DIM), jnp.float32),
            pltpu.VMEM((CCH, DIM), jnp.float32),
            pltpu.VMEM((CCH, DIM), jnp.float32),
            pltpu.VMEM((BPW,), jnp.float32),
            pltpu.SemaphoreType.DMA,
        ],
        compiler_params=pltpu.CompilerParams(
            needs_layout_passes=False,
        ),
    )(_tile_kernel)
    out = f(heads, rels, tails, ent, rel)
    dh = jnp.take(ent, heads, axis=0)
    dt = jnp.take(ent, tails, axis=0)
    return out + (dh[:, 0] + dt[:, 0]) * jnp.float32(0.0)


def kernel(heads, relations, tails, entity_emb, relation_emb):
    heads = jnp.asarray(heads, jnp.int32)
    relations = jnp.asarray(relations, jnp.int32)
    tails = jnp.asarray(tails, jnp.int32)
    return _transe_sc(heads, relations, tails, entity_emb, relation_emb)


# speedup vs baseline: 1.3952x; 1.0716x over previous
import functools

import jax
import jax.numpy as jnp
from jax import lax
from jax.experimental import pallas as pl
from jax.experimental.pallas import tpu as pltpu
from jax.experimental.pallas import tpu_sc as plsc
from jax.experimental.layout import Layout, with_layout_constraint

BATCH = 16384
DIM = 64
NC = 2
NS = 16
NW = NC * NS
BPW = BATCH // NW
L = 16
NG = BPW // L


def _rsqrt16(x):
    i = plsc.bitcast(x, jnp.int32)
    i = jnp.int32(0x5F3759DF) - lax.shift_right_arithmetic(i, jnp.int32(1))
    y = plsc.bitcast(i, jnp.float32)
    xh = x * jnp.float32(0.5)
    for _ in range(3):
        y = y * (jnp.float32(1.5) - xh * y * y)
    return y


CCH = 256
NCC = BPW // CCH
NGC = CCH // L


def _tile_kernel(heads, rels, tails, ent, rel, out,
                 ih, ir, it, hr, rr, tr, sc, sem):
    wid = lax.axis_index("s") * NC + lax.axis_index("c")
    base = wid * BPW

    pltpu.sync_copy(heads.at[pl.ds(base, BPW)], ih)
    pltpu.sync_copy(rels.at[pl.ds(base, BPW)], ir)
    pltpu.sync_copy(tails.at[pl.ds(base, BPW)], it)

    iota = lax.iota(jnp.int32, L)
    zero = jnp.zeros((L,), jnp.float32)

    for ck in range(NCC):
        def fetch(g, ck=ck):
            rb = g * L
            gb = ck * CCH + rb
            vh = ih[pl.ds(gb, L)]
            vr = ir[pl.ds(gb, L)]
            vt = it[pl.ds(gb, L)]
            copies = []
            for k in range(L):
                dst = (pl.ds(rb + k, 1), pl.ds(0, DIM))
                copies.append(pltpu.async_copy(
                    ent.at[pl.ds(vh[k], 1), :], hr.at[dst[0], dst[1]], sem))
                copies.append(pltpu.async_copy(
                    rel.at[pl.ds(vr[k], 1), :], rr.at[dst[0], dst[1]], sem))
                copies.append(pltpu.async_copy(
                    ent.at[pl.ds(vt[k], 1), :], tr.at[dst[0], dst[1]], sem))
            return copies

        def group(g, _, ck=ck):
            rowi = g * L + iota

            sh0 = zero
            sh1 = zero
            st0 = zero
            st1 = zero
            for j in range(0, DIM, 2):
                c0 = jnp.full((L,), j, jnp.int32)
                c1 = jnp.full((L,), j + 1, jnp.int32)
                hv0 = plsc.load_gather(hr, [rowi, c0])
                hv1 = plsc.load_gather(hr, [rowi, c1])
                tv0 = plsc.load_gather(tr, [rowi, c0])
                tv1 = plsc.load_gather(tr, [rowi, c1])
                sh0 = sh0 + hv0 * hv0
                sh1 = sh1 + hv1 * hv1
                st0 = st0 + tv0 * tv0
                st1 = st1 + tv1 * tv1
            invh = _rsqrt16(sh0 + sh1)
            invt = _rsqrt16(st0 + st1)

            a0 = zero
            a1 = zero
            for j in range(0, DIM, 2):
                c0 = jnp.full((L,), j, jnp.int32)
                c1 = jnp.full((L,), j + 1, jnp.int32)
                hv0 = plsc.load_gather(hr, [rowi, c0])
                hv1 = plsc.load_gather(hr, [rowi, c1])
                tv0 = plsc.load_gather(tr, [rowi, c0])
                tv1 = plsc.load_gather(tr, [rowi, c1])
                rv0 = plsc.load_gather(rr, [rowi, c0])
                rv1 = plsc.load_gather(rr, [rowi, c1])
                a0 = a0 + jnp.abs(hv0 * invh + rv0 - tv0 * invt)
                a1 = a1 + jnp.abs(hv1 * invh + rv1 - tv1 * invt)
            sc[pl.ds(ck * CCH + g * L, L)] = a0 + a1
            return 0

        def step(g, _, ck=ck):
            copies = fetch(g, ck=ck)

            @pl.when(g > 0)
            def _():
                group(g - 1, None, ck=ck)

            for cp in copies:
                cp.wait()
            return 0

        lax.fori_loop(0, NGC, step, 0)
        group(NGC - 1, None, ck=ck)

    pltpu.sync_copy(sc, out.at[pl.ds(base, BPW)])


@jax.jit
def _transe_sc(heads, rels, tails, ent, rel):
    ent = with_layout_constraint(
        ent, Layout(major_to_minor=(1, 0), tiling=((8, 128),)))
    rel = with_layout_constraint(
        rel, Layout(major_to_minor=(1, 0), tiling=((8, 128),)))
    mesh = plsc.VectorSubcoreMesh(core_axis_name="c", subcore_axis_name="s")
    f = functools.partial(
        pl.kernel,
        mesh=mesh,
        out_type=jax.ShapeDtypeStruct((BATCH,), jnp.float32),
        scratch_types=[
            pltpu.VMEM((BPW,), jnp.int32),
            pltpu.VMEM((BPW,), jnp.int32),
            pltpu.VMEM((BPW,), jnp.int32),
            pltpu.VMEM((CCH, DIM), jnp.float32),
            pltpu.VMEM((CCH, DIM), jnp.float32),
            pltpu.VMEM((CCH, DIM), jnp.float32),
            pltpu.VMEM((BPW,), jnp.float32),
            pltpu.SemaphoreType.DMA,
        ],
        compiler_params=pltpu.CompilerParams(
            needs_layout_passes=False,
        ),
    )(_tile_kernel)
    return f(heads, rels, tails, ent, rel)


def kernel(heads, relations, tails, entity_emb, relation_emb):
    heads = jnp.asarray(heads, jnp.int32)
    relations = jnp.asarray(relations, jnp.int32)
    tails = jnp.asarray(tails, jnp.int32)
    return _transe_sc(heads, relations, tails, entity_emb, relation_emb)
